# baseline (speedup 1.0000x reference)
import jax
import jax.numpy as jnp
from jax.experimental import pallas as pl
from jax.experimental.pallas import tpu as pltpu

NS = 6


def _body(pos_ref, hid_ref, mask_ref, emb_ref, out_ref, mask_out_ref, sem, csem):
    b, k = pl.program_id(0), pl.program_id(1)
    nct = emb_ref.shape[1]
    osb = out_ref.shape[1]

    @pl.when(k == 0)
    def _():
        ctx = pltpu.make_async_copy(
            emb_ref.at[pos_ref[0]], out_ref.at[0, :nct], csem
        )
        ctx.start()
        bulk = pltpu.make_async_copy(
            hid_ref.at[b, pl.ds(0, osb - nct)],
            out_ref.at[0, pl.ds(nct, osb - nct)],
            sem,
        )
        bulk.start()
        mask_out_ref[0, 0, :nct] = jnp.ones((nct,), mask_out_ref.dtype)
        mask_out_ref[0, 0, nct:] = mask_ref[0, 0]
        ctx.wait()
        bulk.wait()

    @pl.when(k > 0)
    def _():
        bulk = pltpu.make_async_copy(
            hid_ref.at[b, pl.ds(k * osb - nct, osb)], out_ref.at[0], sem
        )
        bulk.start()
        bulk.wait()


def kernel(hidden_states, attention_mask, embeddings, position):
    B, S, D = hidden_states.shape
    _, NCT, _ = embeddings.shape
    pos = jnp.asarray(position, jnp.int32).reshape((1,))
    osb = (NCT + S) // NS
    hid4 = hidden_states.reshape(B, S, 8, D // 8)
    emb4 = embeddings.reshape(embeddings.shape[0], NCT, 8, D // 8)
    mask3 = attention_mask.reshape(B, 1, S)

    grid_spec = pltpu.PrefetchScalarGridSpec(
        num_scalar_prefetch=1,
        grid=(B, NS),
        in_specs=[
            pl.BlockSpec(memory_space=pl.ANY),
            pl.BlockSpec((1, 1, S), lambda b, k, p: (b, 0, 0)),
            pl.BlockSpec(memory_space=pl.ANY),
        ],
        out_specs=[
            pl.BlockSpec((1, osb, 8, D // 8), lambda b, k, p: (b, k, 0, 0)),
            pl.BlockSpec((1, 1, NCT + S), lambda b, k, p: (b, 0, 0)),
        ],
        scratch_shapes=[pltpu.SemaphoreType.DMA, pltpu.SemaphoreType.DMA],
    )

    out_hid, out_mask = pl.pallas_call(
        _body,
        grid_spec=grid_spec,
        out_shape=[
            jax.ShapeDtypeStruct((B, NCT + S, 8, D // 8), hidden_states.dtype),
            jax.ShapeDtypeStruct((B, 1, NCT + S), attention_mask.dtype),
        ],
    )(pos, hid4, mask3, emb4)
    return (out_hid.reshape(B, NCT + S, D), out_mask.reshape(B, NCT + S))


# speedup vs baseline: 1.1451x; 1.1451x over previous
import jax
import jax.numpy as jnp
from jax.experimental import pallas as pl
from jax.experimental.pallas import tpu as pltpu

NS = 6


def _body(pos_ref, hid_ref, mask_ref, emb_ref, out_ref, mask_out_ref, carry_ref):
    k = pl.program_id(1)
    nct = emb_ref.shape[1]
    sb = out_ref.shape[1]

    @pl.when(k == 0)
    def _():
        out_ref[0, :nct] = emb_ref[0]
        mask_out_ref[0, 0, :nct] = jnp.ones((nct,), mask_out_ref.dtype)
        mask_out_ref[0, 0, nct:] = mask_ref[0, 0]

    @pl.when(k > 0)
    def _():
        out_ref[0, :nct] = carry_ref[:]

    out_ref[0, nct:] = hid_ref[0, : sb - nct]
    carry_ref[:] = hid_ref[0, sb - nct :]


def kernel(hidden_states, attention_mask, embeddings, position):
    B, S, D = hidden_states.shape
    _, NCT, _ = embeddings.shape
    pos = jnp.asarray(position, jnp.int32).reshape((1,))
    sb = (NCT + S) // NS
    hid4 = hidden_states.reshape(B, S, 8, D // 8)
    emb4 = embeddings.reshape(embeddings.shape[0], NCT, 8, D // 8)
    mask3 = attention_mask.reshape(B, 1, S)

    grid_spec = pltpu.PrefetchScalarGridSpec(
        num_scalar_prefetch=1,
        grid=(B, NS),
        in_specs=[
            pl.BlockSpec((1, sb, 8, D // 8), lambda b, k, p: (b, k, 0, 0)),
            pl.BlockSpec((1, 1, S), lambda b, k, p: (b, 0, 0)),
            pl.BlockSpec((1, NCT, 8, D // 8), lambda b, k, p: (p[0], 0, 0, 0)),
        ],
        out_specs=[
            pl.BlockSpec((1, sb, 8, D // 8), lambda b, k, p: (b, k, 0, 0)),
            pl.BlockSpec((1, 1, NCT + S), lambda b, k, p: (b, 0, 0)),
        ],
        scratch_shapes=[pltpu.VMEM((NCT, 8, D // 8), hidden_states.dtype)],
    )

    out_hid, out_mask = pl.pallas_call(
        _body,
        grid_spec=grid_spec,
        out_shape=[
            jax.ShapeDtypeStruct((B, NCT + S, 8, D // 8), hidden_states.dtype),
            jax.ShapeDtypeStruct((B, 1, NCT + S), attention_mask.dtype),
        ],
    )(pos, hid4, mask3, emb4)
    return (out_hid.reshape(B, NCT + S, D), out_mask.reshape(B, NCT + S))
